# Initial kernel scaffold; baseline (speedup 1.0000x reference)
import jax
import jax.numpy as jnp
from jax.experimental import pallas as pl


def kernel(x, pos, edge_index, edge_attr, node_sigma_emb, We1, be1, We2, be2, Wf1_0, bf1_0, Wf2_0, bf2_0, Wtp_0, Wf1_1, bf1_1, Wf2_1, bf2_1, Wtp_1):
    raise NotImplementedError("write your pallas kernel here")


# speedup vs baseline: 2.2346x; 2.2346x over previous
import functools

import jax
import jax.numpy as jnp
from jax import lax
from jax.experimental import pallas as pl
from jax.experimental.pallas import tpu as pltpu
from jax.experimental.pallas import tpu_sc as plsc

N_NODES = 50000
N_EDGES = 800000
NS = 16
NP = 50016
HALF = NP // 2
SC_CORES = 2
SC_TILES = 16
NW = SC_CORES * SC_TILES

_S3 = 3.0 ** 0.5
_S5 = 5.0 ** 0.5
_S15 = 15.0 ** 0.5


def _sc_mesh():
    return plsc.VectorSubcoreMesh(
        core_axis_name="c", subcore_axis_name="s",
        num_cores=SC_CORES, num_subcores=SC_TILES)


def _make_gather2(na, da, nb, db):
    per_w = N_EDGES // NW
    C = 128
    n_main = per_w // C
    tail = per_w - n_main * C

    @functools.partial(
        pl.kernel,
        out_type=(jax.ShapeDtypeStruct((N_EDGES, da), jnp.float32),
                  jax.ShapeDtypeStruct((N_EDGES, db), jnp.float32)),
        mesh=_sc_mesh(),
        compiler_params=pltpu.CompilerParams(use_tc_tiling_on_sc=False),
        scratch_types=[
            pltpu.VMEM((C,), jnp.int32), pltpu.VMEM((C,), jnp.int32),
            pltpu.VMEM((C, da), jnp.float32), pltpu.VMEM((C, db), jnp.float32),
            pltpu.VMEM((tail,), jnp.int32), pltpu.VMEM((tail,), jnp.int32),
            pltpu.VMEM((tail, da), jnp.float32), pltpu.VMEM((tail, db), jnp.float32),
            pltpu.SemaphoreType.DMA, pltpu.SemaphoreType.DMA,
        ],
    )
    def gather2(ta, tb, ia, ib, oa, ob,
                ia_v, ib_v, ra_v, rb_v, ia_t, ib_t, ra_t, rb_t, sem_a, sem_b):
        wid = lax.axis_index("s") * SC_CORES + lax.axis_index("c")
        base = wid * per_w

        def step(off, iav, ibv, rav, rbv):
            n = iav.shape[0]
            pltpu.sync_copy(ia.at[pl.ds(off, n)], iav)
            pltpu.sync_copy(ib.at[pl.ds(off, n)], ibv)
            ca = pltpu.async_copy(ta.at[iav], rav, sem_a)
            cb = pltpu.async_copy(tb.at[ibv], rbv, sem_b)
            ca.wait()
            cb.wait()
            pltpu.sync_copy(rav, oa.at[pl.ds(off, n)])
            pltpu.sync_copy(rbv, ob.at[pl.ds(off, n)])

        def body(i, carry):
            step(base + i * C, ia_v, ib_v, ra_v, rb_v)
            return carry

        lax.fori_loop(0, n_main, body, 0)
        if tail:
            step(base + n_main * C, ia_t, ib_t, ra_t, rb_t)

    return gather2


def _make_scatter(d):
    per_t = N_EDGES // SC_TILES
    C = 128
    n_main = per_t // C
    tail = per_t - n_main * C
    rows_pt = HALF // SC_TILES

    @functools.partial(
        pl.kernel,
        out_type=jax.ShapeDtypeStruct((NP, d), jnp.float32),
        mesh=_sc_mesh(),
        compiler_params=pltpu.CompilerParams(use_tc_tiling_on_sc=False),
        scratch_types=[
            pltpu.VMEM_SHARED((HALF + 16, d), jnp.float32),
            pltpu.VMEM((C,), jnp.int32), pltpu.VMEM((C,), jnp.int32),
            pltpu.VMEM((C, d), jnp.float32),
            pltpu.VMEM((tail,), jnp.int32), pltpu.VMEM((tail,), jnp.int32),
            pltpu.VMEM((tail, d), jnp.float32),
        ],
    )
    def scatter(m, dst, hprev, out, acc,
                dst_v, loc_v, m_v, dst_t, loc_t, m_t):
        c = lax.axis_index("c")
        s = lax.axis_index("s")
        row0 = s * rows_pt
        nbase = c * HALF
        pltpu.sync_copy(hprev.at[pl.ds(nbase + row0, rows_pt)],
                        acc.at[pl.ds(row0, rows_pt)])
        plsc.subcore_barrier()

        def step(off, dv, lv, mv):
            n = dv.shape[0]
            pltpu.sync_copy(dst.at[pl.ds(off, n)], dv)
            pltpu.sync_copy(m.at[pl.ds(off, n)], mv)
            for i in range(n // 16):
                v = dv[pl.ds(i * 16, 16)]
                loc = v - nbase
                ok = (loc >= 0) & (loc < HALF)
                dummy = HALF + (v & 15)
                lv[pl.ds(i * 16, 16)] = jnp.where(ok, loc, dummy)
            pltpu.sync_copy(mv, acc.at[lv], add=True)

        ebase = s * per_t

        def body(i, carry):
            step(ebase + i * C, dst_v, loc_v, m_v)
            return carry

        lax.fori_loop(0, n_main, body, 0)
        if tail:
            step(ebase + n_main * C, dst_t, loc_t, m_t)
        plsc.subcore_barrier()

        pltpu.sync_copy(acc.at[pl.ds(row0, rows_pt)],
                        out.at[pl.ds(nbase + row0, rows_pt)])

    return scatter


_TCB = 2000


def _tc0_body(ea, asrc, adst, w1a, w1s, w1d, be1, we2, be2,
              wfa, wfb, wfc, bf1, wf2, bf2, wtp,
              ee_o, sh_o, m_o):
    asrc_v = asrc[...]
    adst_v = adst[...]
    ps = asrc_v[:, 0:4]
    pd = adst_v[:, 0:4]
    ev = pd - ps
    ss = jnp.sum(ev * ev, axis=1, keepdims=True)
    dist = jnp.sqrt(ss + 1e-12)
    offs = lax.broadcasted_iota(jnp.int32, (1, 32), 1).astype(jnp.float32) * (5.0 / 31.0)
    coeff = -0.5 / (5.0 / 31.0) ** 2
    de = jnp.exp(coeff * (dist - offs) ** 2)
    sig = asrc_v[:, 4:36]
    e1 = (jnp.dot(ea[...], w1a[...], preferred_element_type=jnp.float32)
          + jnp.dot(sig, w1s[...], preferred_element_type=jnp.float32)
          + jnp.dot(de, w1d[...], preferred_element_type=jnp.float32)
          + be1[...])
    ee = jnp.dot(jnp.maximum(e1, 0.0), we2[...],
                 preferred_element_type=jnp.float32) + be2[...]

    n = ev / jnp.sqrt(ss + 1e-8)
    xn = n[:, 0:1]
    yn = n[:, 1:2]
    zn = n[:, 2:3]
    sh = jnp.concatenate([
        jnp.ones_like(xn), _S3 * xn, _S3 * yn, _S3 * zn,
        _S15 * xn * yn, _S15 * yn * zn, 0.5 * _S5 * (3.0 * zn * zn - 1.0),
        _S15 * xn * zn, 0.5 * _S15 * (xn * xn - yn * yn),
        jnp.zeros_like(xn[:, :1].repeat(7, axis=1)),
    ], axis=1)

    xs = asrc_v[:, 36:52]
    xd = adst_v[:, 4:20]
    f1 = (jnp.dot(ee, wfa[...], preferred_element_type=jnp.float32)
          + jnp.dot(xs, wfb[...], preferred_element_type=jnp.float32)
          + jnp.dot(xd, wfc[...], preferred_element_type=jnp.float32)
          + bf1[...])
    gate = jnp.dot(jnp.maximum(f1, 0.0), wf2[...],
                   preferred_element_type=jnp.float32) + bf2[...]
    t = gate * xs
    m = jnp.zeros((t.shape[0], m_o.shape[1]), jnp.float32)
    for j in range(9):
        m = m + jnp.dot(t * sh[:, j:j + 1], wtp[pl.ds(j * 16, 16), :],
                        preferred_element_type=jnp.float32)
    ee_o[...] = ee
    sh_o[...] = sh
    m_o[...] = m * 0.25


def _tc1_body(ee, sh, h1s, h1d, wfa, wfb, wfc, bf1, wf2, bf2, wtp, m_o):
    ee_v = ee[...]
    sh_v = sh[...]
    h1s_v = h1s[...]
    f1 = (jnp.dot(ee_v, wfa[...], preferred_element_type=jnp.float32)
          + jnp.dot(h1s_v[:, 0:16], wfb[...], preferred_element_type=jnp.float32)
          + jnp.dot(h1d[...], wfc[...], preferred_element_type=jnp.float32)
          + bf1[...])
    gate = jnp.dot(jnp.maximum(f1, 0.0), wf2[...],
                   preferred_element_type=jnp.float32) + bf2[...]
    t = gate * h1s_v
    m = jnp.zeros((t.shape[0], m_o.shape[1]), jnp.float32)
    for j in range(9):
        m = m + jnp.dot(t * sh_v[:, j:j + 1], wtp[pl.ds(j * 32, 32), :],
                        preferred_element_type=jnp.float32)
    m_o[...] = m * 0.25


def _edge_spec(cols):
    return pl.BlockSpec((_TCB, cols), lambda i: (i, 0))


def _full_spec(shape):
    nd = len(shape)
    return pl.BlockSpec(shape, lambda i, _n=nd: (0,) * _n)


def _arrange_wtp(wtp, din, dout, din_p, dout_p):
    w = wtp.reshape(din, 9, dout).transpose(1, 0, 2)
    w = jnp.pad(w, ((0, 0), (0, din_p - din), (0, dout_p - dout)))
    return w.reshape(9 * din_p, dout_p)


def kernel(x, pos, edge_index, edge_attr, node_sigma_emb, We1, be1, We2, be2,
           Wf1_0, bf1_0, Wf2_0, bf2_0, Wtp_0,
           Wf1_1, bf1_1, Wf2_1, bf2_1, Wtp_1):
    src = edge_index[0]
    dst = edge_index[1]
    pos4 = jnp.pad(pos, ((0, 0), (0, 1)))
    ts = jnp.pad(jnp.concatenate([pos4, node_sigma_emb, x], axis=1),
                 ((0, 0), (0, 12)))
    td = jnp.pad(jnp.concatenate([pos4, x], axis=1),
                 ((0, 0), (0, 12)))

    a_src, a_dst = _make_gather2(N_NODES, 64, N_NODES, 32)(ts, td, src, dst)

    grid = N_EDGES // _TCB
    ee, sh, m0 = pl.pallas_call(
        _tc0_body,
        grid=(grid,),
        in_specs=[
            _edge_spec(4), _edge_spec(64), _edge_spec(32),
            _full_spec((4, 16)), _full_spec((32, 16)), _full_spec((32, 16)),
            _full_spec((1, 16)), _full_spec((16, 16)), _full_spec((1, 16)),
            _full_spec((16, 48)), _full_spec((16, 48)), _full_spec((16, 48)),
            _full_spec((1, 48)), _full_spec((48, 16)), _full_spec((1, 16)),
            _full_spec((144, 32)),
        ],
        out_specs=[_edge_spec(16), _edge_spec(16), _edge_spec(32)],
        out_shape=[
            jax.ShapeDtypeStruct((N_EDGES, 16), jnp.float32),
            jax.ShapeDtypeStruct((N_EDGES, 16), jnp.float32),
            jax.ShapeDtypeStruct((N_EDGES, 32), jnp.float32),
        ],
    )(edge_attr, a_src, a_dst,
      We1[0:4], We1[4:36], We1[36:68], be1[None, :], We2, be2[None, :],
      Wf1_0[0:16], Wf1_0[16:32], Wf1_0[32:48], bf1_0[None, :],
      Wf2_0, bf2_0[None, :],
      _arrange_wtp(Wtp_0, 16, 28, 16, 32))

    hp0 = jnp.pad(x, ((0, NP - N_NODES), (0, 16)))
    h1 = _make_scatter(32)(m0, dst, hp0)

    h1_16 = h1[:N_NODES, :16]
    h1s, h1d = _make_gather2(NP, 32, N_NODES, 16)(h1, h1_16, src, dst)

    m1 = pl.pallas_call(
        _tc1_body,
        grid=(grid,),
        in_specs=[
            _edge_spec(16), _edge_spec(16), _edge_spec(32), _edge_spec(16),
            _full_spec((16, 48)), _full_spec((16, 48)), _full_spec((16, 48)),
            _full_spec((1, 48)), _full_spec((48, 32)), _full_spec((1, 32)),
            _full_spec((288, 48)),
        ],
        out_specs=[_edge_spec(48)],
        out_shape=[jax.ShapeDtypeStruct((N_EDGES, 48), jnp.float32)],
    )(ee, sh, h1s, h1d,
      Wf1_1[0:16], Wf1_1[16:32], Wf1_1[32:48], bf1_1[None, :],
      jnp.pad(Wf2_1, ((0, 0), (0, 4))), jnp.pad(bf2_1, (0, 4))[None, :],
      _arrange_wtp(Wtp_1, 28, 40, 32, 48))[0]

    hp1 = jnp.pad(h1, ((0, 0), (0, 16)))
    h2 = _make_scatter(48)(m1, dst, hp1)
    return h2[:N_NODES, :40]


# speedup vs baseline: 2.5597x; 1.1455x over previous
import functools

import jax
import jax.numpy as jnp
import numpy as np
from jax import lax
from jax.experimental import pallas as pl
from jax.experimental.pallas import tpu as pltpu
from jax.experimental.pallas import tpu_sc as plsc

N_NODES = 50000
N_EDGES = 800000
NS = 16
NP = 50016
HALF = NP // 2
SC_CORES = 2
SC_TILES = 16
NW = SC_CORES * SC_TILES

_S3 = 3.0 ** 0.5
_S5 = 5.0 ** 0.5
_S15 = 15.0 ** 0.5


def _sc_mesh():
    return plsc.VectorSubcoreMesh(
        core_axis_name="c", subcore_axis_name="s",
        num_cores=SC_CORES, num_subcores=SC_TILES)


def _make_gather2(na, da, nb, db):
    per_w = N_EDGES // NW
    C = 128
    n_main = per_w // C
    tail = per_w - n_main * C

    @functools.partial(
        pl.kernel,
        out_type=(jax.ShapeDtypeStruct((N_EDGES, da), jnp.float32),
                  jax.ShapeDtypeStruct((N_EDGES, db), jnp.float32)),
        mesh=_sc_mesh(),
        compiler_params=pltpu.CompilerParams(use_tc_tiling_on_sc=False),
        scratch_types=[
            pltpu.VMEM((C,), jnp.int32), pltpu.VMEM((C,), jnp.int32),
            pltpu.VMEM((C, da), jnp.float32), pltpu.VMEM((C, db), jnp.float32),
            pltpu.VMEM((tail,), jnp.int32), pltpu.VMEM((tail,), jnp.int32),
            pltpu.VMEM((tail, da), jnp.float32), pltpu.VMEM((tail, db), jnp.float32),
            pltpu.SemaphoreType.DMA, pltpu.SemaphoreType.DMA,
        ],
    )
    def gather2(ta, tb, ia, ib, oa, ob,
                ia_v, ib_v, ra_v, rb_v, ia_t, ib_t, ra_t, rb_t, sem_a, sem_b):
        wid = lax.axis_index("s") * SC_CORES + lax.axis_index("c")
        base = wid * per_w

        def step(off, iav, ibv, rav, rbv):
            n = iav.shape[0]
            pltpu.sync_copy(ia.at[pl.ds(off, n)], iav)
            pltpu.sync_copy(ib.at[pl.ds(off, n)], ibv)
            ca = pltpu.async_copy(ta.at[iav], rav, sem_a)
            cb = pltpu.async_copy(tb.at[ibv], rbv, sem_b)
            ca.wait()
            cb.wait()
            pltpu.sync_copy(rav, oa.at[pl.ds(off, n)])
            pltpu.sync_copy(rbv, ob.at[pl.ds(off, n)])

        def body(i, carry):
            step(base + i * C, ia_v, ib_v, ra_v, rb_v)
            return carry

        lax.fori_loop(0, n_main, body, 0)
        if tail:
            step(base + n_main * C, ia_t, ib_t, ra_t, rb_t)

    return gather2


def _make_scatter(d):
    per_t = N_EDGES // SC_TILES
    C = 128
    n_main = per_t // C
    tail = per_t - n_main * C
    rows_pt = HALF // SC_TILES

    @functools.partial(
        pl.kernel,
        out_type=jax.ShapeDtypeStruct((NP, d), jnp.float32),
        mesh=_sc_mesh(),
        compiler_params=pltpu.CompilerParams(use_tc_tiling_on_sc=False),
        scratch_types=[
            pltpu.VMEM_SHARED((HALF + 16, d), jnp.float32),
            pltpu.VMEM((C,), jnp.int32), pltpu.VMEM((C,), jnp.int32),
            pltpu.VMEM((C, d), jnp.float32),
            pltpu.VMEM((tail,), jnp.int32), pltpu.VMEM((tail,), jnp.int32),
            pltpu.VMEM((tail, d), jnp.float32),
        ],
    )
    def scatter(m, dst, hprev, out, acc,
                dst_v, loc_v, m_v, dst_t, loc_t, m_t):
        c = lax.axis_index("c")
        s = lax.axis_index("s")
        row0 = s * rows_pt
        nbase = c * HALF
        pltpu.sync_copy(hprev.at[pl.ds(nbase + row0, rows_pt)],
                        acc.at[pl.ds(row0, rows_pt)])
        plsc.subcore_barrier()

        def step(off, dv, lv, mv):
            n = dv.shape[0]
            pltpu.sync_copy(dst.at[pl.ds(off, n)], dv)
            pltpu.sync_copy(m.at[pl.ds(off, n)], mv)
            for i in range(n // 16):
                v = dv[pl.ds(i * 16, 16)]
                loc = v - nbase
                ok = (loc >= 0) & (loc < HALF)
                dummy = HALF + (v & 15)
                lv[pl.ds(i * 16, 16)] = jnp.where(ok, loc, dummy)
            pltpu.sync_copy(mv, acc.at[lv], add=True)

        ebase = s * per_t

        def body(i, carry):
            step(ebase + i * C, dst_v, loc_v, m_v)
            return carry

        lax.fori_loop(0, n_main, body, 0)
        if tail:
            step(ebase + n_main * C, dst_t, loc_t, m_t)
        plsc.subcore_barrier()

        pltpu.sync_copy(acc.at[pl.ds(row0, rows_pt)],
                        out.at[pl.ds(nbase + row0, rows_pt)])

    return scatter


_TCB = 2000


def _mm(a, b):
    return jnp.dot(a, b, preferred_element_type=jnp.float32)


def _tc0_body(ea, asrc, adst, ones44, spread432, offs, w1a, w1s, w1d, be1,
              we2, be2, wfa, wfb, wfc, bf1, wf2, bf2,
              c1, c2, c3, cc, rx, rg, wtp,
              ee_o, n4_o, m_o):
    asrc_v = asrc[...]
    adst_v = adst[...]
    ev = adst_v[:, 0:4] - asrc_v[:, 0:4]
    ss4 = _mm(ev * ev, ones44[...])
    d4 = jnp.sqrt(ss4 + 1e-12)
    n4 = ev * lax.rsqrt(ss4 + 1e-8)
    d32 = _mm(d4, spread432[...])
    coeff = -0.5 / (5.0 / 31.0) ** 2
    de = jnp.exp(coeff * (d32 - offs[...]) ** 2)
    e1 = _mm(ea[...], w1a[...]) + _mm(asrc_v, w1s[...]) + _mm(de, w1d[...]) + be1[...]
    ee = _mm(jnp.maximum(e1, 0.0), we2[...]) + be2[...]

    f1 = (_mm(ee, wfa[...]) + _mm(asrc_v, wfb[...]) + _mm(adst_v, wfc[...])
          + bf1[...])
    gate = _mm(jnp.maximum(f1, 0.0), wf2[...]) + bf2[...]
    sh_rep = _mm(n4, c1[...]) * _mm(n4, c2[...]) + _mm(n4, c3[...]) + cc[...]
    p = _mm(asrc_v, rx[...]) * _mm(gate, rg[...]) * sh_rep
    ee_o[...] = ee
    n4_o[...] = n4
    m_o[...] = _mm(p, wtp[...]) * 0.25


def _tc1_body(ee, n4, h1s, h1d, wfa, wfb, wfc, bf1, wf2, bf2,
              c1, c2, c3, cc, rh, rg, wtp, m_o):
    n4_v = n4[...]
    h1s_v = h1s[...]
    f1 = (_mm(ee[...], wfa[...]) + _mm(h1s_v, wfb[...]) + _mm(h1d[...], wfc[...])
          + bf1[...])
    gate = _mm(jnp.maximum(f1, 0.0), wf2[...]) + bf2[...]
    sh_rep = _mm(n4_v, c1[...]) * _mm(n4_v, c2[...]) + _mm(n4_v, c3[...]) + cc[...]
    p = _mm(h1s_v, rh[...]) * _mm(gate, rg[...]) * sh_rep
    m_o[...] = _mm(p, wtp[...]) * 0.25


def _edge_spec(cols):
    return pl.BlockSpec((_TCB, cols), lambda i: (i, 0))


def _full_spec(shape):
    nd = len(shape)
    return pl.BlockSpec(shape, lambda i, _n=nd: (0,) * _n)


def _arrange_wtp(wtp, din, dout, dout_p):
    w = wtp.reshape(din, 9, dout).transpose(1, 0, 2)
    w = jnp.pad(w, ((0, 0), (0, 0), (0, dout_p - dout)))
    return w.reshape(9 * din, dout_p)


def _sh_mats(din):
    c1 = np.zeros((4, 9), np.float32)
    c2 = np.zeros((4, 9), np.float32)
    c3 = np.zeros((4, 9), np.float32)
    cc = np.zeros((9,), np.float32)
    cc[0] = 1.0
    c3[0, 1] = _S3
    c3[1, 2] = _S3
    c3[2, 3] = _S3
    c1[0, 4] = 1.0
    c2[1, 4] = _S15
    c1[1, 5] = 1.0
    c2[2, 5] = _S15
    c1[2, 6] = 1.0
    c2[2, 6] = 1.5 * _S5
    cc[6] = -0.5 * _S5
    c1[0, 7] = 1.0
    c2[2, 7] = _S15
    c1[0, 8] = 1.0
    c1[1, 8] = -1.0
    c2[0, 8] = 0.5 * _S15
    c2[1, 8] = 0.5 * _S15
    rep = lambda m: jnp.asarray(np.repeat(m, din, axis=1))
    return rep(c1), rep(c2), rep(c3), jnp.asarray(np.repeat(cc, din)[None, :])


def _rep_eye(rows, din, off=0):
    r = np.zeros((rows, 9 * din), np.float32)
    for j in range(9):
        for k in range(din):
            r[off + k, j * din + k] = 1.0
    return jnp.asarray(r)


def kernel(x, pos, edge_index, edge_attr, node_sigma_emb, We1, be1, We2, be2,
           Wf1_0, bf1_0, Wf2_0, bf2_0, Wtp_0,
           Wf1_1, bf1_1, Wf2_1, bf2_1, Wtp_1):
    src = edge_index[0]
    dst = edge_index[1]
    pos4 = jnp.pad(pos, ((0, 0), (0, 1)))
    ts = jnp.pad(jnp.concatenate([pos4, node_sigma_emb, x], axis=1),
                 ((0, 0), (0, 12)))
    td = jnp.pad(jnp.concatenate([pos4, x], axis=1),
                 ((0, 0), (0, 12)))

    a_src, a_dst = _make_gather2(N_NODES, 64, N_NODES, 32)(ts, td, src, dst)

    c1_0, c2_0, c3_0, cc_0 = _sh_mats(16)
    ones44 = jnp.ones((4, 4), jnp.float32)
    spread432 = jnp.full((4, 32), 0.25, jnp.float32)
    offs = (jnp.arange(32, dtype=jnp.float32) * (5.0 / 31.0))[None, :]

    grid = N_EDGES // _TCB
    ee, n4, m0 = pl.pallas_call(
        _tc0_body,
        grid=(grid,),
        in_specs=[
            _edge_spec(4), _edge_spec(64), _edge_spec(32),
            _full_spec((4, 4)), _full_spec((4, 32)), _full_spec((1, 32)),
            _full_spec((4, 16)), _full_spec((64, 16)), _full_spec((32, 16)),
            _full_spec((1, 16)), _full_spec((16, 16)), _full_spec((1, 16)),
            _full_spec((16, 48)), _full_spec((64, 48)), _full_spec((32, 48)),
            _full_spec((1, 48)), _full_spec((48, 16)), _full_spec((1, 16)),
            _full_spec((4, 144)), _full_spec((4, 144)), _full_spec((4, 144)),
            _full_spec((1, 144)), _full_spec((64, 144)), _full_spec((16, 144)),
            _full_spec((144, 32)),
        ],
        out_specs=[_edge_spec(16), _edge_spec(4), _edge_spec(32)],
        out_shape=[
            jax.ShapeDtypeStruct((N_EDGES, 16), jnp.float32),
            jax.ShapeDtypeStruct((N_EDGES, 4), jnp.float32),
            jax.ShapeDtypeStruct((N_EDGES, 32), jnp.float32),
        ],
    )(edge_attr, a_src, a_dst, ones44, spread432, offs,
      We1[0:4], jnp.pad(We1[4:36], ((4, 28), (0, 0))), We1[36:68],
      be1[None, :], We2, be2[None, :],
      Wf1_0[0:16], jnp.pad(Wf1_0[16:32], ((36, 12), (0, 0))),
      jnp.pad(Wf1_0[32:48], ((4, 12), (0, 0))), bf1_0[None, :],
      Wf2_0, bf2_0[None, :],
      c1_0, c2_0, c3_0, cc_0, _rep_eye(64, 16, 36), _rep_eye(16, 16),
      _arrange_wtp(Wtp_0, 16, 28, 32))

    hp0 = jnp.pad(x, ((0, NP - N_NODES), (0, 16)))
    h1 = _make_scatter(32)(m0, dst, hp0)

    h1_16 = h1[:N_NODES, :16]
    h1s, h1d = _make_gather2(NP, 32, N_NODES, 16)(h1, h1_16, src, dst)

    c1_1, c2_1, c3_1, cc_1 = _sh_mats(28)
    m1 = pl.pallas_call(
        _tc1_body,
        grid=(grid,),
        in_specs=[
            _edge_spec(16), _edge_spec(4), _edge_spec(32), _edge_spec(16),
            _full_spec((16, 48)), _full_spec((32, 48)), _full_spec((16, 48)),
            _full_spec((1, 48)), _full_spec((48, 28)), _full_spec((1, 28)),
            _full_spec((4, 252)), _full_spec((4, 252)), _full_spec((4, 252)),
            _full_spec((1, 252)), _full_spec((32, 252)), _full_spec((28, 252)),
            _full_spec((252, 48)),
        ],
        out_specs=[_edge_spec(48)],
        out_shape=[jax.ShapeDtypeStruct((N_EDGES, 48), jnp.float32)],
    )(ee, n4, h1s, h1d,
      Wf1_1[0:16], jnp.pad(Wf1_1[16:32], ((0, 16), (0, 0))), Wf1_1[32:48],
      bf1_1[None, :], Wf2_1, bf2_1[None, :],
      c1_1, c2_1, c3_1, cc_1, _rep_eye(32, 28), _rep_eye(28, 28),
      _arrange_wtp(Wtp_1, 28, 40, 48))[0]

    hp1 = jnp.pad(h1, ((0, 0), (0, 16)))
    h2 = _make_scatter(48)(m1, dst, hp1)
    return h2[:N_NODES, :40]


# speedup vs baseline: 3.0778x; 1.2024x over previous
import functools

import jax
import jax.numpy as jnp
import numpy as np
from jax import lax
from jax.experimental import pallas as pl
from jax.experimental.pallas import tpu as pltpu
from jax.experimental.pallas import tpu_sc as plsc

N_NODES = 50000
N_EDGES = 800000
NS = 16
NP = 50016
HALF = NP // 2
SC_CORES = 2
SC_TILES = 16
NW = SC_CORES * SC_TILES

_S3 = 3.0 ** 0.5
_S5 = 5.0 ** 0.5
_S15 = 15.0 ** 0.5


def _sc_mesh():
    return plsc.VectorSubcoreMesh(
        core_axis_name="c", subcore_axis_name="s",
        num_cores=SC_CORES, num_subcores=SC_TILES)


def _make_gather2(na, nb):
    per_w = N_EDGES // NW
    C = 128
    n_main = per_w // C
    tail = per_w - n_main * C

    @functools.partial(
        pl.kernel,
        out_type=jax.ShapeDtypeStruct((N_EDGES, 128), jnp.float32),
        mesh=_sc_mesh(),
        compiler_params=pltpu.CompilerParams(use_tc_tiling_on_sc=False),
        scratch_types=[
            pltpu.VMEM((C,), jnp.int32), pltpu.VMEM((C,), jnp.int32),
            pltpu.VMEM((C, 64), jnp.float32), pltpu.VMEM((C, 64), jnp.float32),
            pltpu.VMEM((tail,), jnp.int32), pltpu.VMEM((tail,), jnp.int32),
            pltpu.VMEM((tail, 64), jnp.float32), pltpu.VMEM((tail, 64), jnp.float32),
            pltpu.SemaphoreType.DMA, pltpu.SemaphoreType.DMA,
        ],
    )
    def gather2(ta, tb, ia, ib, oc,
                ia_v, ib_v, ra_v, rb_v, ia_t, ib_t, ra_t, rb_t, sem_a, sem_b):
        wid = lax.axis_index("s") * SC_CORES + lax.axis_index("c")
        base = wid * per_w

        def step(off, iav, ibv, rav, rbv):
            n = iav.shape[0]
            pltpu.sync_copy(ia.at[pl.ds(off, n)], iav)
            pltpu.sync_copy(ib.at[pl.ds(off, n)], ibv)
            ca = pltpu.async_copy(ta.at[iav], rav, sem_a)
            cb = pltpu.async_copy(tb.at[ibv], rbv, sem_b)
            ca.wait()
            cb.wait()
            pltpu.sync_copy(rav, oc.at[pl.ds(off, n), pl.ds(0, 64)])
            pltpu.sync_copy(rbv, oc.at[pl.ds(off, n), pl.ds(64, 64)])

        def body(i, carry):
            step(base + i * C, ia_v, ib_v, ra_v, rb_v)
            return carry

        lax.fori_loop(0, n_main, body, 0)
        if tail:
            step(base + n_main * C, ia_t, ib_t, ra_t, rb_t)

    return gather2


def _make_scatter(d):
    per_t = N_EDGES // SC_TILES
    C = 128
    n_main = per_t // C
    tail = per_t - n_main * C
    rows_pt = HALF // SC_TILES

    @functools.partial(
        pl.kernel,
        out_type=jax.ShapeDtypeStruct((NP, d), jnp.float32),
        mesh=_sc_mesh(),
        compiler_params=pltpu.CompilerParams(use_tc_tiling_on_sc=False),
        scratch_types=[
            pltpu.VMEM_SHARED((HALF + 16, d), jnp.float32),
            pltpu.VMEM((C,), jnp.int32), pltpu.VMEM((C,), jnp.int32),
            pltpu.VMEM((C, d), jnp.float32),
            pltpu.VMEM((tail,), jnp.int32), pltpu.VMEM((tail,), jnp.int32),
            pltpu.VMEM((tail, d), jnp.float32),
        ],
    )
    def scatter(m, dst, hprev, out, acc,
                dst_v, loc_v, m_v, dst_t, loc_t, m_t):
        c = lax.axis_index("c")
        s = lax.axis_index("s")
        row0 = s * rows_pt
        nbase = c * HALF
        pltpu.sync_copy(hprev.at[pl.ds(nbase + row0, rows_pt)],
                        acc.at[pl.ds(row0, rows_pt)])
        plsc.subcore_barrier()

        def step(off, dv, lv, mv):
            n = dv.shape[0]
            pltpu.sync_copy(dst.at[pl.ds(off, n)], dv)
            pltpu.sync_copy(m.at[pl.ds(off, n)], mv)
            for i in range(n // 16):
                v = dv[pl.ds(i * 16, 16)]
                loc = v - nbase
                ok = (loc >= 0) & (loc < HALF)
                dummy = HALF + (v & 15)
                lv[pl.ds(i * 16, 16)] = jnp.where(ok, loc, dummy)
            pltpu.sync_copy(mv, acc.at[lv], add=True)

        ebase = s * per_t

        def body(i, carry):
            step(ebase + i * C, dst_v, loc_v, m_v)
            return carry

        lax.fori_loop(0, n_main, body, 0)
        if tail:
            step(ebase + n_main * C, dst_t, loc_t, m_t)
        plsc.subcore_barrier()

        pltpu.sync_copy(acc.at[pl.ds(row0, rows_pt)],
                        out.at[pl.ds(nbase + row0, rows_pt)])

    return scatter


_TCB = 2000


def _mm(a, b):
    return jnp.dot(a, b, preferred_element_type=jnp.float32)


def _tc0_body(ea, ab, ones44, spread432, offs, e3, w1a, w1s, w1d, be1,
              we2, be2, wfa, wfbc, bf1, wf2rg,
              c1, c2, cc, rx, wtp,
              ee_o, n4_o, m_o):
    ab_v = ab[...]
    ev = ab_v[:, 64:68] - ab_v[:, 0:4]
    ss4 = _mm(ev * ev, ones44[...])
    d4 = jnp.sqrt(ss4 + 1e-12)
    n4 = ev * lax.rsqrt(ss4 + 1e-8) + e3[...]
    d32 = _mm(d4, spread432[...])
    coeff = -0.5 / (5.0 / 31.0) ** 2
    de = jnp.exp(coeff * (d32 - offs[...]) ** 2)
    e1 = _mm(ea[...], w1a[...]) + _mm(ab_v, w1s[...]) + _mm(de, w1d[...]) + be1[...]
    ee = _mm(jnp.maximum(e1, 0.0), we2[...]) + be2[...]

    f1 = _mm(ee, wfa[...]) + _mm(ab_v, wfbc[...]) + bf1[...]
    g_rep = _mm(jnp.maximum(f1, 0.0), wf2rg[...])
    sh_rep = _mm(n4, c1[...]) * _mm(n4, c2[...]) + cc[...]
    p = _mm(ab_v, rx[...]) * g_rep * sh_rep
    ee_o[...] = ee
    n4_o[...] = n4
    m_o[...] = _mm(p, wtp[...]) * 0.25


def _tc1_body(ee, n4, ab, wfa, wfbc, bf1, wf2rg, c1, c2, cc, rh, wtp, m_o):
    n4_v = n4[...]
    ab_v = ab[...]
    f1 = _mm(ee[...], wfa[...]) + _mm(ab_v, wfbc[...]) + bf1[...]
    g_rep = _mm(jnp.maximum(f1, 0.0), wf2rg[...])
    sh_rep = _mm(n4_v, c1[...]) * _mm(n4_v, c2[...]) + cc[...]
    p = _mm(ab_v, rh[...]) * g_rep * sh_rep
    m_o[...] = _mm(p, wtp[...]) * 0.25


def _edge_spec(cols):
    return pl.BlockSpec((_TCB, cols), lambda i: (i, 0))


def _full_spec(shape):
    nd = len(shape)
    return pl.BlockSpec(shape, lambda i, _n=nd: (0,) * _n)


def _arrange_wtp(wtp, din, dout, dout_p):
    w = wtp.reshape(din, 9, dout).transpose(1, 0, 2)
    w = jnp.pad(w, ((0, 0), (0, 0), (0, dout_p - dout)))
    return w.reshape(9 * din, dout_p)


def _sh_mats(din):
    c1 = np.zeros((4, 9), np.float32)
    c2 = np.zeros((4, 9), np.float32)
    cc = np.zeros((9,), np.float32)
    c1[3, 0] = 1.0
    c2[3, 0] = 1.0
    c1[0, 1] = 1.0
    c2[3, 1] = _S3
    c1[1, 2] = 1.0
    c2[3, 2] = _S3
    c1[2, 3] = 1.0
    c2[3, 3] = _S3
    c1[0, 4] = 1.0
    c2[1, 4] = _S15
    c1[1, 5] = 1.0
    c2[2, 5] = _S15
    c1[2, 6] = 1.0
    c2[2, 6] = 1.5 * _S5
    cc[6] = -0.5 * _S5
    c1[0, 7] = 1.0
    c2[2, 7] = _S15
    c1[0, 8] = 1.0
    c1[1, 8] = -1.0
    c2[0, 8] = 0.5 * _S15
    c2[1, 8] = 0.5 * _S15
    rep = lambda m: jnp.asarray(np.repeat(m, din, axis=1))
    return rep(c1), rep(c2), jnp.asarray(np.repeat(cc, din)[None, :])


def _rep_eye(rows, din, off=0):
    r = np.zeros((rows, 9 * din), np.float32)
    for j in range(9):
        for k in range(din):
            r[off + k, j * din + k] = 1.0
    return jnp.asarray(r)


def kernel(x, pos, edge_index, edge_attr, node_sigma_emb, We1, be1, We2, be2,
           Wf1_0, bf1_0, Wf2_0, bf2_0, Wtp_0,
           Wf1_1, bf1_1, Wf2_1, bf2_1, Wtp_1):
    src = edge_index[0]
    dst = edge_index[1]
    pos4 = jnp.pad(pos, ((0, 0), (0, 1)))
    ts = jnp.pad(jnp.concatenate([pos4, node_sigma_emb, x], axis=1),
                 ((0, 0), (0, 12)))
    td = jnp.pad(jnp.concatenate([pos4, x], axis=1),
                 ((0, 0), (0, 44)))

    ab0 = _make_gather2(N_NODES, N_NODES)(ts, td, src, dst)

    c1_0, c2_0, cc_0 = _sh_mats(16)
    ones44 = jnp.ones((4, 4), jnp.float32)
    spread432 = jnp.full((4, 32), 0.25, jnp.float32)
    offs = (jnp.arange(32, dtype=jnp.float32) * (5.0 / 31.0))[None, :]
    e3 = jnp.asarray(np.array([[0.0, 0.0, 0.0, 1.0]], np.float32))

    def rowpad(w, lo, width=128):
        return jnp.pad(w, ((lo, width - lo - w.shape[0]), (0, 0)))

    grid = N_EDGES // _TCB
    ee, n4, m0 = pl.pallas_call(
        _tc0_body,
        grid=(grid,),
        in_specs=[
            _edge_spec(4), _edge_spec(128),
            _full_spec((4, 4)), _full_spec((4, 32)), _full_spec((1, 32)),
            _full_spec((1, 4)),
            _full_spec((4, 16)), _full_spec((128, 16)), _full_spec((32, 16)),
            _full_spec((1, 16)), _full_spec((16, 16)), _full_spec((1, 16)),
            _full_spec((16, 48)), _full_spec((128, 48)), _full_spec((1, 48)),
            _full_spec((48, 144)),
            _full_spec((4, 144)), _full_spec((4, 144)), _full_spec((1, 144)),
            _full_spec((128, 144)), _full_spec((144, 32)),
        ],
        out_specs=[_edge_spec(16), _edge_spec(4), _edge_spec(32)],
        out_shape=[
            jax.ShapeDtypeStruct((N_EDGES, 16), jnp.float32),
            jax.ShapeDtypeStruct((N_EDGES, 4), jnp.float32),
            jax.ShapeDtypeStruct((N_EDGES, 32), jnp.float32),
        ],
    )(edge_attr, ab0, ones44, spread432, offs, e3,
      We1[0:4], rowpad(We1[4:36], 4), We1[36:68],
      be1[None, :], We2, be2[None, :],
      Wf1_0[0:16], rowpad(Wf1_0[16:32], 36) + rowpad(Wf1_0[32:48], 68),
      bf1_0[None, :], jnp.dot(Wf2_0, _rep_eye(16, 16)),
      c1_0, c2_0, cc_0, _rep_eye(128, 16, 36),
      _arrange_wtp(Wtp_0, 16, 28, 32))

    hp0 = jnp.pad(x, ((0, NP - N_NODES), (0, 16)))
    h1 = _make_scatter(32)(m0, dst, hp0)

    h1_16 = h1[:N_NODES, :16]
    ab1 = _make_gather2(NP, N_NODES)(
        jnp.pad(h1, ((0, 0), (0, 32))),
        jnp.pad(h1_16, ((0, 0), (0, 48))), src, dst)

    c1_1, c2_1, cc_1 = _sh_mats(28)
    m1 = pl.pallas_call(
        _tc1_body,
        grid=(grid,),
        in_specs=[
            _edge_spec(16), _edge_spec(4), _edge_spec(128),
            _full_spec((16, 48)), _full_spec((128, 48)), _full_spec((1, 48)),
            _full_spec((48, 252)),
            _full_spec((4, 252)), _full_spec((4, 252)), _full_spec((1, 252)),
            _full_spec((128, 252)), _full_spec((252, 48)),
        ],
        out_specs=[_edge_spec(48)],
        out_shape=[jax.ShapeDtypeStruct((N_EDGES, 48), jnp.float32)],
    )(ee, n4, ab1,
      Wf1_1[0:16], rowpad(Wf1_1[16:32], 0) + rowpad(Wf1_1[32:48], 64),
      bf1_1[None, :], jnp.dot(Wf2_1, _rep_eye(28, 28)),
      c1_1, c2_1, cc_1, _rep_eye(128, 28),
      _arrange_wtp(Wtp_1, 28, 40, 48))[0]

    hp1 = jnp.pad(h1, ((0, 0), (0, 16)))
    h2 = _make_scatter(48)(m1, dst, hp1)
    return h2[:N_NODES, :40]
